# Initial kernel scaffold; baseline (speedup 1.0000x reference)
import functools

import jax
import jax.numpy as jnp
from jax import lax
from jax.experimental import pallas as pl
from jax.experimental.pallas import tpu as pltpu
from jax.experimental.pallas import tpu_sc as plsc

N = 10000
E = 320000
D_IN = 128
D_OUT = 32
D2 = 2 * D_OUT

NC = 2
NS = 16
NW = NC * NS
EPW = E // NW
B = 80
NB = EPW // B

ROWS_MAIN = 632
ROWS_LAST = N - (NS - 1) * ROWS_MAIN

R = 2000
G = N // R

_mesh = plsc.VectorSubcoreMesh(core_axis_name="c", subcore_axis_name="s")


def _node_slice_copy(copy_fn, sid):
    @pl.when(sid != NS - 1)
    def _():
        copy_fn(sid * ROWS_MAIN, ROWS_MAIN)

    @pl.when(sid == NS - 1)
    def _():
        copy_fn((NS - 1) * ROWS_MAIN, ROWS_LAST)


@functools.partial(
    pl.kernel,
    out_type=jax.ShapeDtypeStruct((NC, N), jnp.float32),
    mesh=_mesh,
    scratch_types=[
        pltpu.VMEM((B,), jnp.int32),
        pltpu.VMEM((B,), jnp.float32),
        pltpu.VMEM_SHARED((N,), jnp.float32),
    ],
)
def _deg_kernel(dst_hbm, zeros_hbm, out_hbm, dst_v, ones_v, acc_sh):
    cid = lax.axis_index("c")
    sid = lax.axis_index("s")
    wid = sid * NC + cid

    _node_slice_copy(
        lambda s, n: pltpu.sync_copy(zeros_hbm.at[pl.ds(s, n)],
                                     acc_sh.at[pl.ds(s, n)]), sid)
    for i in range(B // 16):
        ones_v[pl.ds(i * 16, 16)] = jnp.ones((16,), jnp.float32)
    plsc.subcore_barrier()

    def body(i, carry):
        base = wid * EPW + i * B
        pltpu.sync_copy(dst_hbm.at[pl.ds(base, B)], dst_v)
        pltpu.sync_copy(ones_v, acc_sh.at[dst_v], add=True)
        return carry

    lax.fori_loop(0, NB, body, 0)
    plsc.subcore_barrier()
    _node_slice_copy(
        lambda s, n: pltpu.sync_copy(acc_sh.at[pl.ds(s, n)],
                                     out_hbm.at[cid, pl.ds(s, n)]), sid)


@functools.partial(
    pl.kernel,
    out_type=jax.ShapeDtypeStruct((NC, N, D2), jnp.float32),
    mesh=_mesh,
    scratch_types=[
        pltpu.VMEM((B,), jnp.int32),
        pltpu.VMEM((B,), jnp.int32),
        pltpu.VMEM((B, D2), jnp.float32),
        pltpu.VMEM_SHARED((N, D2), jnp.float32),
        pltpu.SemaphoreType.DMA,
    ],
)
def _prop_kernel(src_hbm, dst_hbm, feat_hbm, zeros_hbm, out_hbm,
                 src_v, dst_v, rows_v, acc_sh, sem):
    cid = lax.axis_index("c")
    sid = lax.axis_index("s")
    wid = sid * NC + cid

    _node_slice_copy(
        lambda s, n: pltpu.sync_copy(zeros_hbm.at[pl.ds(s, n)],
                                     acc_sh.at[pl.ds(s, n)]), sid)
    plsc.subcore_barrier()

    def body(i, carry):
        base = wid * EPW + i * B
        pltpu.sync_copy(src_hbm.at[pl.ds(base, B)], src_v)
        pltpu.sync_copy(dst_hbm.at[pl.ds(base, B)], dst_v)
        pltpu.async_copy(feat_hbm.at[src_v], rows_v, sem).wait()
        pltpu.sync_copy(rows_v, acc_sh.at[dst_v], add=True)
        return carry

    lax.fori_loop(0, NB, body, 0)
    plsc.subcore_barrier()
    _node_slice_copy(
        lambda s, n: pltpu.sync_copy(acc_sh.at[pl.ds(s, n)],
                                     out_hbm.at[cid, pl.ds(s, n)]), sid)


def _feat_body(x_ref, w_ref, b_ref, o_ref):
    y = jnp.dot(x_ref[...], w_ref[...],
                preferred_element_type=jnp.float32) + b_ref[...]
    xa = y[:, :D_OUT]
    h = y[:, D_OUT:]
    nrm = jnp.sqrt(jnp.sum(h * h, axis=1, keepdims=True))
    h = h * (1.8 / jnp.maximum(nrm, 1e-12))
    o_ref[...] = jnp.concatenate([xa, h], axis=1)


_feat_call = pl.pallas_call(
    _feat_body,
    grid=(G,),
    in_specs=[
        pl.BlockSpec((R, D_IN), lambda i: (i, 0)),
        pl.BlockSpec((D_IN, D2), lambda i: (0, 0)),
        pl.BlockSpec((1, D2), lambda i: (0, 0)),
    ],
    out_specs=pl.BlockSpec((R, D2), lambda i: (i, 0)),
    out_shape=jax.ShapeDtypeStruct((N, D2), jnp.float32),
)


def _scale_body(degT_ref, feat_ref, dsq_ref, fs_ref):
    d = degT_ref[:, 0:1] + degT_ref[:, 1:2]
    dsq = lax.rsqrt(d)
    dsq_ref[...] = dsq
    fs_ref[...] = feat_ref[...] * dsq


_scale_call = pl.pallas_call(
    _scale_body,
    grid=(G,),
    in_specs=[
        pl.BlockSpec((R, 2), lambda i: (i, 0)),
        pl.BlockSpec((R, D2), lambda i: (i, 0)),
    ],
    out_specs=[
        pl.BlockSpec((R, 1), lambda i: (i, 0)),
        pl.BlockSpec((R, D2), lambda i: (i, 0)),
    ],
    out_shape=[
        jax.ShapeDtypeStruct((N, 1), jnp.float32),
        jax.ShapeDtypeStruct((N, D2), jnp.float32),
    ],
)


def _combine_body(a0_ref, a1_ref, fs_ref, dsq_ref, ox_ref, oh_ref):
    t = (a0_ref[...] + a1_ref[...] + fs_ref[...]) * dsq_ref[...]
    ox_ref[...] = t[:, :D_OUT]
    oh_ref[...] = t[:, D_OUT:]


_combine_call = pl.pallas_call(
    _combine_body,
    grid=(G,),
    in_specs=[
        pl.BlockSpec((R, D2), lambda i: (i, 0)),
        pl.BlockSpec((R, D2), lambda i: (i, 0)),
        pl.BlockSpec((R, D2), lambda i: (i, 0)),
        pl.BlockSpec((R, 1), lambda i: (i, 0)),
    ],
    out_specs=[
        pl.BlockSpec((R, D_OUT), lambda i: (i, 0)),
        pl.BlockSpec((R, D_OUT), lambda i: (i, 0)),
    ],
    out_shape=[
        jax.ShapeDtypeStruct((N, D_OUT), jnp.float32),
        jax.ShapeDtypeStruct((N, D_OUT), jnp.float32),
    ],
)


def kernel(x, edge_index, W1, b1, W2, b2):
    src = edge_index[0]
    dst = edge_index[1]
    W = jnp.concatenate([W1, W2], axis=1)
    b = jnp.concatenate([b1, b2])[None, :]

    feat = _feat_call(x, W, b)
    deg = _deg_kernel(dst, jnp.zeros((N,), jnp.float32))
    dsq, fs = _scale_call(deg.T, feat)
    acc = _prop_kernel(src, dst, fs, jnp.zeros((N, D2), jnp.float32))
    ox, oh = _combine_call(acc[0], acc[1], fs, dsq)
    return (oh, ox)


# speedup vs baseline: 22.9973x; 22.9973x over previous
import functools

import jax
import jax.numpy as jnp
from jax import lax
from jax.experimental import pallas as pl
from jax.experimental.pallas import tpu as pltpu
from jax.experimental.pallas import tpu_sc as plsc

N = 10000
E = 320000
D_IN = 128
D_OUT = 32
D2 = 2 * D_OUT

NC = 2
NS = 16
NW = NC * NS
EPW = E // NW
B = 80
NB = EPW // B

ROWS_MAIN = 632
ROWS_LAST = N - (NS - 1) * ROWS_MAIN

R = 2000
G = N // R

_mesh = plsc.VectorSubcoreMesh(core_axis_name="c", subcore_axis_name="s")


def _node_slice_copy(copy_fn, sid):
    @pl.when(sid != NS - 1)
    def _():
        copy_fn(sid * ROWS_MAIN, ROWS_MAIN)

    @pl.when(sid == NS - 1)
    def _():
        copy_fn((NS - 1) * ROWS_MAIN, ROWS_LAST)


@functools.partial(
    pl.kernel,
    out_type=jax.ShapeDtypeStruct((NC * N,), jnp.float32),
    mesh=_mesh,
    scratch_types=[
        pltpu.VMEM((B,), jnp.int32),
        pltpu.VMEM((B,), jnp.float32),
        pltpu.VMEM((ROWS_MAIN,), jnp.float32),
        pltpu.VMEM_SHARED((N,), jnp.float32),
    ],
)
def _deg_kernel(dst_hbm, zeros_hbm, out_hbm, dst_v, ones_v, stage_v, acc_sh):
    cid = lax.axis_index("c")
    sid = lax.axis_index("s")
    wid = sid * NC + cid

    pltpu.sync_copy(zeros_hbm, stage_v)
    _node_slice_copy(
        lambda s, n: pltpu.sync_copy(stage_v.at[pl.ds(0, n)],
                                     acc_sh.at[pl.ds(s, n)]), sid)
    for i in range(B // 16):
        ones_v[pl.ds(i * 16, 16)] = jnp.ones((16,), jnp.float32)
    plsc.subcore_barrier()

    def body(i, carry):
        base = wid * EPW + i * B
        pltpu.sync_copy(dst_hbm.at[pl.ds(base, B)], dst_v)
        pltpu.sync_copy(ones_v, acc_sh.at[dst_v], add=True)
        return carry

    lax.fori_loop(0, NB, body, 0)
    plsc.subcore_barrier()

    def _writeout(s, n):
        pltpu.sync_copy(acc_sh.at[pl.ds(s, n)], stage_v.at[pl.ds(0, n)])
        pltpu.sync_copy(stage_v.at[pl.ds(0, n)],
                        out_hbm.at[pl.ds(cid * N + s, n)])

    _node_slice_copy(_writeout, sid)


@functools.partial(
    pl.kernel,
    out_type=jax.ShapeDtypeStruct((NC * N, D2), jnp.float32),
    mesh=_mesh,
    scratch_types=[
        pltpu.VMEM((B,), jnp.int32),
        pltpu.VMEM((B,), jnp.int32),
        pltpu.VMEM((B, D2), jnp.float32),
        pltpu.VMEM((ROWS_MAIN, D2), jnp.float32),
        pltpu.VMEM_SHARED((N, D2), jnp.float32),
        pltpu.SemaphoreType.DMA,
    ],
    compiler_params=pltpu.CompilerParams(use_tc_tiling_on_sc=False),
)
def _prop_kernel(src_hbm, dst_hbm, feat_hbm, zeros_hbm, out_hbm,
                 src_v, dst_v, rows_v, stage_v, acc_sh, sem):
    cid = lax.axis_index("c")
    sid = lax.axis_index("s")
    wid = sid * NC + cid

    pltpu.sync_copy(zeros_hbm, stage_v)
    _node_slice_copy(
        lambda s, n: pltpu.sync_copy(stage_v.at[pl.ds(0, n)],
                                     acc_sh.at[pl.ds(s, n)]), sid)
    plsc.subcore_barrier()

    def body(i, carry):
        base = wid * EPW + i * B
        pltpu.sync_copy(src_hbm.at[pl.ds(base, B)], src_v)
        pltpu.sync_copy(dst_hbm.at[pl.ds(base, B)], dst_v)
        pltpu.async_copy(feat_hbm.at[src_v], rows_v, sem).wait()
        pltpu.sync_copy(rows_v, acc_sh.at[dst_v], add=True)
        return carry

    lax.fori_loop(0, NB, body, 0)
    plsc.subcore_barrier()

    def _writeout(s, n):
        pltpu.sync_copy(acc_sh.at[pl.ds(s, n)], stage_v.at[pl.ds(0, n)])
        pltpu.sync_copy(stage_v.at[pl.ds(0, n)],
                        out_hbm.at[pl.ds(cid * N + s, n)])

    _node_slice_copy(_writeout, sid)


def _feat_body(x_ref, w_ref, b_ref, o_ref):
    y = jnp.dot(x_ref[...], w_ref[...],
                preferred_element_type=jnp.float32) + b_ref[...]
    xa = y[:, :D_OUT]
    h = y[:, D_OUT:]
    nrm = jnp.sqrt(jnp.sum(h * h, axis=1, keepdims=True))
    h = h * (1.8 / jnp.maximum(nrm, 1e-12))
    o_ref[...] = jnp.concatenate([xa, h], axis=1)


_feat_call = pl.pallas_call(
    _feat_body,
    grid=(G,),
    in_specs=[
        pl.BlockSpec((R, D_IN), lambda i: (i, 0)),
        pl.BlockSpec((D_IN, D2), lambda i: (0, 0)),
        pl.BlockSpec((1, D2), lambda i: (0, 0)),
    ],
    out_specs=pl.BlockSpec((R, D2), lambda i: (i, 0)),
    out_shape=jax.ShapeDtypeStruct((N, D2), jnp.float32),
)


def _scale_body(degT_ref, feat_ref, dsq_ref, fs_ref):
    d = degT_ref[:, 0:1] + degT_ref[:, 1:2] + 1.0
    dsq = lax.rsqrt(d)
    dsq_ref[...] = dsq
    fs_ref[...] = feat_ref[...] * dsq


_scale_call = pl.pallas_call(
    _scale_body,
    grid=(G,),
    in_specs=[
        pl.BlockSpec((R, 2), lambda i: (i, 0)),
        pl.BlockSpec((R, D2), lambda i: (i, 0)),
    ],
    out_specs=[
        pl.BlockSpec((R, 1), lambda i: (i, 0)),
        pl.BlockSpec((R, D2), lambda i: (i, 0)),
    ],
    out_shape=[
        jax.ShapeDtypeStruct((N, 1), jnp.float32),
        jax.ShapeDtypeStruct((N, D2), jnp.float32),
    ],
)


def _combine_body(a0_ref, a1_ref, fs_ref, dsq_ref, ox_ref, oh_ref):
    t = (a0_ref[...] + a1_ref[...] + fs_ref[...]) * dsq_ref[...]
    ox_ref[...] = t[:, :D_OUT]
    oh_ref[...] = t[:, D_OUT:]


_combine_call = pl.pallas_call(
    _combine_body,
    grid=(G,),
    in_specs=[
        pl.BlockSpec((R, D2), lambda i: (i, 0)),
        pl.BlockSpec((R, D2), lambda i: (i, 0)),
        pl.BlockSpec((R, D2), lambda i: (i, 0)),
        pl.BlockSpec((R, 1), lambda i: (i, 0)),
    ],
    out_specs=[
        pl.BlockSpec((R, D_OUT), lambda i: (i, 0)),
        pl.BlockSpec((R, D_OUT), lambda i: (i, 0)),
    ],
    out_shape=[
        jax.ShapeDtypeStruct((N, D_OUT), jnp.float32),
        jax.ShapeDtypeStruct((N, D_OUT), jnp.float32),
    ],
)


def kernel(x, edge_index, W1, b1, W2, b2):
    src = edge_index[0]
    dst = edge_index[1]
    W = jnp.concatenate([W1, W2], axis=1)
    b = jnp.concatenate([b1, b2])[None, :]

    feat = _feat_call(x, W, b)
    deg = _deg_kernel(dst, jnp.zeros((ROWS_MAIN,), jnp.float32)).reshape(NC, N)
    dsq, fs = _scale_call(deg.T, feat)
    acc = _prop_kernel(src, dst, fs,
                       jnp.zeros((ROWS_MAIN, D2), jnp.float32)).reshape(NC, N, D2)
    ox, oh = _combine_call(acc[0], acc[1], fs, dsq)
    return (oh, ox)


# speedup vs baseline: 23.9158x; 1.0399x over previous
import functools

import jax
import jax.numpy as jnp
from jax import lax
from jax.experimental import pallas as pl
from jax.experimental.pallas import tpu as pltpu
from jax.experimental.pallas import tpu_sc as plsc

N = 10000
E = 320000
D_IN = 128
D_OUT = 32
D2 = 2 * D_OUT

NC = 2
NS = 16
NW = NC * NS
B = 128
NB = 80
EPW = NB * B
E_PAD = NW * EPW
NA = N + 8

ROWS_MAIN = 632
ROWS_LAST = N - (NS - 1) * ROWS_MAIN

R = 2000
G = N // R

_mesh = plsc.VectorSubcoreMesh(core_axis_name="c", subcore_axis_name="s")
_sc_params = pltpu.CompilerParams(use_tc_tiling_on_sc=False)


def _node_slice_copy(copy_fn, sid):
    @pl.when(sid != NS - 1)
    def _():
        copy_fn(sid * ROWS_MAIN, ROWS_MAIN)

    @pl.when(sid == NS - 1)
    def _():
        copy_fn((NS - 1) * ROWS_MAIN, ROWS_LAST)


@functools.partial(
    pl.kernel,
    out_type=jax.ShapeDtypeStruct((NC * N,), jnp.float32),
    mesh=_mesh,
    scratch_types=[
        pltpu.VMEM((NB, B), jnp.int32),
        pltpu.VMEM((B,), jnp.float32),
        pltpu.VMEM((ROWS_MAIN,), jnp.float32),
        pltpu.VMEM_SHARED((NA,), jnp.float32),
        pltpu.SemaphoreType.DMA,
    ],
    compiler_params=_sc_params,
)
def _deg_kernel(dst_hbm, zeros_hbm, out_hbm, dst_v, ones_v, stage_v,
                acc_sh, sem):
    cid = lax.axis_index("c")
    sid = lax.axis_index("s")
    wid = sid * NC + cid

    pltpu.sync_copy(zeros_hbm, stage_v)
    _node_slice_copy(
        lambda s, n: pltpu.sync_copy(stage_v.at[pl.ds(0, n)],
                                     acc_sh.at[pl.ds(s, n)]), sid)

    @pl.when(sid == 0)
    def _():
        pltpu.sync_copy(zeros_hbm.at[pl.ds(0, 8)], acc_sh.at[pl.ds(N, 8)])

    pltpu.sync_copy(dst_hbm.at[pl.ds(wid * NB, NB)], dst_v)
    for i in range(B // 16):
        ones_v[pl.ds(i * 16, 16)] = jnp.ones((16,), jnp.float32)
    plsc.subcore_barrier()

    K = 8

    def body(g, carry):
        for j in range(K):
            pltpu.async_copy(ones_v, acc_sh.at[dst_v.at[g * K + j]], sem,
                             add=True)
        for j in range(K):
            pltpu.make_async_copy(ones_v, acc_sh.at[dst_v.at[g * K + j]],
                                  sem).wait()
        return carry

    lax.fori_loop(0, NB // K, body, 0)
    plsc.subcore_barrier()

    def _writeout(s, n):
        pltpu.sync_copy(acc_sh.at[pl.ds(s, n)], stage_v.at[pl.ds(0, n)])
        pltpu.sync_copy(stage_v.at[pl.ds(0, n)],
                        out_hbm.at[pl.ds(cid * N + s, n)])

    _node_slice_copy(_writeout, sid)


@functools.partial(
    pl.kernel,
    out_type=jax.ShapeDtypeStruct((NC * N, D2), jnp.float32),
    mesh=_mesh,
    scratch_types=[
        pltpu.VMEM((NB, B), jnp.int32),
        pltpu.VMEM((NB, B), jnp.int32),
        pltpu.VMEM((B, D2), jnp.float32),
        pltpu.VMEM((B, D2), jnp.float32),
        pltpu.VMEM((ROWS_MAIN, D2), jnp.float32),
        pltpu.VMEM_SHARED((NA, D2), jnp.float32),
        pltpu.SemaphoreType.DMA,
        pltpu.SemaphoreType.DMA,
    ],
    compiler_params=_sc_params,
)
def _prop_kernel(src_hbm, dst_hbm, feat_hbm, zeros_hbm, out_hbm,
                 src_v, dst_v, rows0_v, rows1_v, stage_v, acc_sh,
                 sem0, sem1):
    cid = lax.axis_index("c")
    sid = lax.axis_index("s")
    wid = sid * NC + cid

    pltpu.sync_copy(zeros_hbm, stage_v)
    _node_slice_copy(
        lambda s, n: pltpu.sync_copy(stage_v.at[pl.ds(0, n)],
                                     acc_sh.at[pl.ds(s, n)]), sid)

    @pl.when(sid == 0)
    def _():
        pltpu.sync_copy(zeros_hbm.at[pl.ds(0, 8)], acc_sh.at[pl.ds(N, 8)])

    pltpu.sync_copy(src_hbm.at[pl.ds(wid * NB, NB)], src_v)
    pltpu.sync_copy(dst_hbm.at[pl.ds(wid * NB, NB)], dst_v)
    plsc.subcore_barrier()

    pltpu.async_copy(feat_hbm.at[src_v.at[0]], rows0_v, sem0)

    def body(k, carry):
        i0 = 2 * k
        i1 = 2 * k + 1
        pltpu.async_copy(feat_hbm.at[src_v.at[i1]], rows1_v, sem1)
        pltpu.make_async_copy(feat_hbm.at[src_v.at[i0]], rows0_v,
                              sem0).wait()
        pltpu.sync_copy(rows0_v, acc_sh.at[dst_v.at[i0]], add=True)

        @pl.when(k < NB // 2 - 1)
        def _():
            pltpu.async_copy(feat_hbm.at[src_v.at[i0 + 2]], rows0_v, sem0)

        pltpu.make_async_copy(feat_hbm.at[src_v.at[i1]], rows1_v,
                              sem1).wait()
        pltpu.sync_copy(rows1_v, acc_sh.at[dst_v.at[i1]], add=True)
        return carry

    lax.fori_loop(0, NB // 2, body, 0)
    plsc.subcore_barrier()

    def _writeout(s, n):
        pltpu.sync_copy(acc_sh.at[pl.ds(s, n)], stage_v.at[pl.ds(0, n)])
        pltpu.sync_copy(stage_v.at[pl.ds(0, n)],
                        out_hbm.at[pl.ds(cid * N + s, n)])

    _node_slice_copy(_writeout, sid)


def _feat_body(x_ref, w_ref, b_ref, o_ref):
    y = jnp.dot(x_ref[...], w_ref[...],
                preferred_element_type=jnp.float32) + b_ref[...]
    xa = y[:, :D_OUT]
    h = y[:, D_OUT:]
    nrm = jnp.sqrt(jnp.sum(h * h, axis=1, keepdims=True))
    h = h * (1.8 / jnp.maximum(nrm, 1e-12))
    o_ref[...] = jnp.concatenate([xa, h], axis=1)


_feat_call = pl.pallas_call(
    _feat_body,
    grid=(G,),
    in_specs=[
        pl.BlockSpec((R, D_IN), lambda i: (i, 0)),
        pl.BlockSpec((D_IN, D2), lambda i: (0, 0)),
        pl.BlockSpec((1, D2), lambda i: (0, 0)),
    ],
    out_specs=pl.BlockSpec((R, D2), lambda i: (i, 0)),
    out_shape=jax.ShapeDtypeStruct((N, D2), jnp.float32),
)


def _scale_body(degT_ref, feat_ref, dsq_ref, fs_ref):
    d = degT_ref[:, 0:1] + degT_ref[:, 1:2] + 1.0
    dsq = lax.rsqrt(d)
    dsq_ref[...] = dsq
    fs_ref[...] = feat_ref[...] * dsq


_scale_call = pl.pallas_call(
    _scale_body,
    grid=(G,),
    in_specs=[
        pl.BlockSpec((R, 2), lambda i: (i, 0)),
        pl.BlockSpec((R, D2), lambda i: (i, 0)),
    ],
    out_specs=[
        pl.BlockSpec((R, 1), lambda i: (i, 0)),
        pl.BlockSpec((R, D2), lambda i: (i, 0)),
    ],
    out_shape=[
        jax.ShapeDtypeStruct((N, 1), jnp.float32),
        jax.ShapeDtypeStruct((N, D2), jnp.float32),
    ],
)


def _combine_body(a0_ref, a1_ref, fs_ref, dsq_ref, ox_ref, oh_ref):
    t = (a0_ref[...] + a1_ref[...] + fs_ref[...]) * dsq_ref[...]
    ox_ref[...] = t[:, :D_OUT]
    oh_ref[...] = t[:, D_OUT:]


_combine_call = pl.pallas_call(
    _combine_body,
    grid=(G,),
    in_specs=[
        pl.BlockSpec((R, D2), lambda i: (i, 0)),
        pl.BlockSpec((R, D2), lambda i: (i, 0)),
        pl.BlockSpec((R, D2), lambda i: (i, 0)),
        pl.BlockSpec((R, 1), lambda i: (i, 0)),
    ],
    out_specs=[
        pl.BlockSpec((R, D_OUT), lambda i: (i, 0)),
        pl.BlockSpec((R, D_OUT), lambda i: (i, 0)),
    ],
    out_shape=[
        jax.ShapeDtypeStruct((N, D_OUT), jnp.float32),
        jax.ShapeDtypeStruct((N, D_OUT), jnp.float32),
    ],
)


def kernel(x, edge_index, W1, b1, W2, b2):
    src = edge_index[0]
    dst = edge_index[1]
    npad = E_PAD - E
    src_p = jnp.concatenate(
        [src, jnp.zeros((npad,), jnp.int32)]).reshape(NW * NB, B)
    dst_p = jnp.concatenate(
        [dst, jnp.full((npad,), N, jnp.int32)]).reshape(NW * NB, B)
    W = jnp.concatenate([W1, W2], axis=1)
    b = jnp.concatenate([b1, b2])[None, :]

    feat = _feat_call(x, W, b)
    deg = _deg_kernel(dst_p, jnp.zeros((ROWS_MAIN,), jnp.float32))
    deg = deg.reshape(NC, N)
    dsq, fs = _scale_call(deg.T, feat)
    acc = _prop_kernel(src_p, dst_p, fs,
                       jnp.zeros((ROWS_MAIN, D2), jnp.float32))
    acc = acc.reshape(NC, N, D2)
    ox, oh = _combine_call(acc[0], acc[1], fs, dsq)
    return (oh, ox)
